# Initial kernel scaffold; baseline (speedup 1.0000x reference)
import jax
import jax.numpy as jnp
from jax.experimental import pallas as pl


def kernel(low8, high8, alignment, small_int, delta, hamming, continuous, bitfield, sketch, low8_table, high8_table, align_table, small_table, delta_table, hamming_table, W_cont, b_cont, W_bits, b_bits, W_sketch, b_sketch, W_val, b_val, W_gate, b_gate, ln_gamma, ln_beta):
    raise NotImplementedError("write your pallas kernel here")


# speedup vs baseline: 3.5048x; 3.5048x over previous
import functools

import jax
import jax.numpy as jnp
from jax import lax
from jax.experimental import pallas as pl
from jax.experimental.pallas import tpu as pltpu
from jax.experimental.pallas import tpu_sc as plsc

D_OUT = 128
EPAD = 8
NW = 32
ROWS_PER_GATHER = 128
CHUNK = 10


def _ternary_fwd(w):
    scale = jnp.mean(jnp.abs(w)) + 1e-5
    return jnp.clip(jnp.round(w / scale), -1.0, 1.0) * scale


@functools.lru_cache(maxsize=None)
def _make_sc_gather(n_tokens):
    per_w = n_tokens // NW
    rows_per_w = per_w // ROWS_PER_GATHER
    n_chunks = rows_per_w // CHUNK
    c_tok = CHUNK * ROWS_PER_GATHER

    mesh = plsc.VectorSubcoreMesh(core_axis_name="c", subcore_axis_name="s")
    out_t = tuple(
        jax.ShapeDtypeStruct((n_tokens, EPAD), jnp.float32) for _ in range(6)
    )

    @functools.partial(
        pl.kernel,
        mesh=mesh,
        out_type=out_t,
        compiler_params=pltpu.CompilerParams(use_tc_tiling_on_sc=False),
        scratch_types=[
            pltpu.VMEM((c_tok,), jnp.int32),
            pltpu.VMEM((c_tok, EPAD), jnp.float32),
            pltpu.SemaphoreType.DMA,
        ],
    )
    def sc_gather(t_low, t_high, t_align, t_small, t_delta, t_ham,
                  i_low, i_high, i_align, i_small, i_delta, i_ham,
                  o_low, o_high, o_align, o_small, o_delta, o_ham,
                  idx_v, rows_v, sem):
        wid = lax.axis_index("c") * 16 + lax.axis_index("s")
        base_tok = wid * per_w

        def stage(table, idx_hbm, out_hbm):
            def body(j, carry):
                pltpu.sync_copy(
                    idx_hbm.at[pl.ds(base_tok + j * c_tok, c_tok)], idx_v)
                cps = []
                for t in range(CHUNK):
                    cps.append(pltpu.async_copy(
                        table.at[idx_v.at[pl.ds(t * ROWS_PER_GATHER,
                                                ROWS_PER_GATHER)]],
                        rows_v.at[pl.ds(t * ROWS_PER_GATHER, ROWS_PER_GATHER)],
                        sem))
                for cp in cps:
                    cp.wait()
                pltpu.sync_copy(
                    rows_v, out_hbm.at[pl.ds(base_tok + j * c_tok, c_tok)])
                return carry

            lax.fori_loop(0, n_chunks, body, 0)

        stage(t_low, i_low, o_low)
        stage(t_high, i_high, o_high)
        stage(t_align, i_align, o_align)
        stage(t_small, i_small, o_small)
        stage(t_delta, i_delta, o_delta)
        stage(t_ham, i_ham, o_ham)

    return sc_gather


def _tc_body(elow, ehigh, eal, esm, edel, eham, cont, bits, sk,
             f_low, f_high, f_al, f_sm, f_del, f_ham, f_cont, f_bits, f_sk,
             bfold, gam, bet, out):
    acc = jnp.dot(elow[...], f_low[...], preferred_element_type=jnp.float32)
    acc += jnp.dot(ehigh[...], f_high[...], preferred_element_type=jnp.float32)
    acc += jnp.dot(eal[...], f_al[...], preferred_element_type=jnp.float32)
    acc += jnp.dot(esm[...], f_sm[...], preferred_element_type=jnp.float32)
    acc += jnp.dot(edel[...], f_del[...], preferred_element_type=jnp.float32)
    acc += jnp.dot(eham[...], f_ham[...], preferred_element_type=jnp.float32)
    acc += jnp.dot(cont[...], f_cont[...], preferred_element_type=jnp.float32)
    acc += jnp.dot(bits[...], f_bits[...], preferred_element_type=jnp.float32)
    acc += jnp.dot(sk[...], f_sk[...], preferred_element_type=jnp.float32)
    acc += bfold[...]
    val = acc[:, :D_OUT]
    gate = jax.nn.sigmoid(acc[:, D_OUT:])
    z = gate * val
    mu = jnp.mean(z, axis=1, keepdims=True)
    zc = z - mu
    var = jnp.mean(zc * zc, axis=1, keepdims=True)
    out[...] = zc / jnp.sqrt(var + 1e-5) * gam[...] + bet[...]


def _tc_call(n_tokens, tn, args):
    grid = (n_tokens // tn,)

    def tok(d):
        return pl.BlockSpec((tn, d), lambda i: (i, 0))

    def full(s):
        return pl.BlockSpec(s, lambda i: (0, 0))

    in_specs = (
        [tok(EPAD)] * 6
        + [tok(6), tok(16), tok(32)]
        + [full((EPAD, 256))] * 6
        + [full((6, 256)), full((16, 256)), full((32, 256))]
        + [full((1, 256)), full((1, D_OUT)), full((1, D_OUT))]
    )
    return pl.pallas_call(
        _tc_body,
        grid=grid,
        in_specs=in_specs,
        out_specs=pl.BlockSpec((tn, D_OUT), lambda i: (i, 0)),
        out_shape=jax.ShapeDtypeStruct((n_tokens, D_OUT), jnp.float32),
        compiler_params=pltpu.CompilerParams(
            dimension_semantics=("arbitrary",)),
    )(*args)


def kernel(low8, high8, alignment, small_int, delta, hamming,
           continuous, bitfield, sketch,
           low8_table, high8_table, align_table, small_table, delta_table,
           hamming_table,
           W_cont, b_cont, W_bits, b_bits, W_sketch, b_sketch,
           W_val, b_val, W_gate, b_gate, ln_gamma, ln_beta):
    B, L = low8.shape
    n = B * L

    def pad8(t):
        d = t.shape[1]
        if d == EPAD:
            return t
        return jnp.pad(t, ((0, 0), (0, EPAD - d)))

    tables = (pad8(low8_table), pad8(high8_table), pad8(align_table),
              pad8(small_table), pad8(delta_table), pad8(hamming_table))
    idxs = tuple(
        a.reshape(n)
        for a in (low8, high8, alignment, small_int, delta, hamming))

    e = _make_sc_gather(n)(*tables, *idxs)

    qc = _ternary_fwd(W_cont)
    qb = _ternary_fwd(W_bits)
    qs = _ternary_fwd(W_sketch)
    Wvg = jnp.concatenate([W_val, W_gate], axis=0)

    def fpad(cols, k):
        f = Wvg[:, cols[0]:cols[1]].T
        d = f.shape[0]
        if d == k:
            return f
        return jnp.pad(f, ((0, k - d), (0, 0)))

    f_low = fpad((0, 8), EPAD)
    f_high = fpad((8, 16), EPAD)
    f_al = fpad((16, 20), EPAD)
    f_sm = fpad((20, 24), EPAD)
    f_del = fpad((24, 30), EPAD)
    f_ham = fpad((30, 34), EPAD)
    f_cont = qc.T @ Wvg[:, 34:42].T
    f_bits = qb.T @ Wvg[:, 42:50].T
    f_sk = qs.T @ Wvg[:, 50:62].T
    bfold = (jnp.concatenate([b_val, b_gate])
             + Wvg[:, 34:42] @ b_cont
             + Wvg[:, 42:50] @ b_bits
             + Wvg[:, 50:62] @ b_sketch)[None, :]

    out = _tc_call(
        n, 1024,
        (*e,
         continuous.reshape(n, 6), bitfield.reshape(n, 16),
         sketch.reshape(n, 32),
         f_low, f_high, f_al, f_sm, f_del, f_ham, f_cont, f_bits, f_sk,
         bfold, ln_gamma[None, :], ln_beta[None, :]))
    return out.reshape(B, L, D_OUT)


# speedup vs baseline: 3.9595x; 1.1297x over previous
import functools

import jax
import jax.numpy as jnp
from jax import lax
from jax.experimental import pallas as pl
from jax.experimental.pallas import tpu as pltpu
from jax.experimental.pallas import tpu_sc as plsc

D_OUT = 128
EPAD = 8
NW = 32
RPG = 128
C_TOK = 640

EMB_FIELDS = ((0, 16), (16, 16), (32, 8), (40, 8), (48, 16), (64, 8))
CONT_COL, BITS_COL, SK_COL = 72, 80, 96


def _ternary_fwd(w):
    scale = jnp.mean(jnp.abs(w)) + 1e-5
    return jnp.clip(jnp.round(w / scale), -1.0, 1.0) * scale


@functools.lru_cache(maxsize=None)
def _make_sc_pack(n_tokens):
    per_w = n_tokens // NW
    n_chunks = per_w // C_TOK
    n_g = C_TOK // RPG

    mesh = plsc.VectorSubcoreMesh(core_axis_name="c", subcore_axis_name="s")

    @functools.partial(
        pl.kernel,
        mesh=mesh,
        out_type=jax.ShapeDtypeStruct((n_tokens, 128), jnp.float32),
        compiler_params=pltpu.CompilerParams(use_tc_tiling_on_sc=False),
        scratch_types=[
            pltpu.VMEM((C_TOK,), jnp.int32),
            pltpu.VMEM((C_TOK, 8), jnp.float32),
            pltpu.VMEM((C_TOK, 16), jnp.float32),
            pltpu.VMEM((C_TOK, 32), jnp.float32),
            pltpu.SemaphoreType.DMA,
        ],
    )
    def sc_pack(t_low, t_high, t_align, t_small, t_delta, t_ham,
                i_low, i_high, i_align, i_small, i_delta, i_ham,
                f_cont, f_bits, f_sk,
                e_all, idx_v, buf8, buf16, buf32, sem):
        wid = lax.axis_index("c") * 16 + lax.axis_index("s")
        base_tok = wid * per_w

        tables = (t_low, t_high, t_align, t_small, t_delta, t_ham)
        idxs = (i_low, i_high, i_align, i_small, i_delta, i_ham)

        def chunk(j, carry):
            tok0 = base_tok + j * C_TOK
            dst = e_all.at[pl.ds(tok0, C_TOK)]
            for table, idx_hbm, (col, width) in zip(tables, idxs, EMB_FIELDS):
                buf = buf16 if width == 16 else buf8
                pltpu.sync_copy(idx_hbm.at[pl.ds(tok0, C_TOK)], idx_v)
                cps = []
                for k in range(n_g):
                    cps.append(pltpu.async_copy(
                        table.at[idx_v.at[pl.ds(k * RPG, RPG)]],
                        buf.at[pl.ds(k * RPG, RPG)], sem))
                for cp in cps:
                    cp.wait()
                pltpu.sync_copy(buf, dst.at[:, pl.ds(col, width)])
            pltpu.sync_copy(f_cont.at[pl.ds(tok0, C_TOK)], buf8)
            pltpu.sync_copy(buf8, dst.at[:, pl.ds(CONT_COL, 8)])
            pltpu.sync_copy(f_bits.at[pl.ds(tok0, C_TOK)], buf16)
            pltpu.sync_copy(buf16, dst.at[:, pl.ds(BITS_COL, 16)])
            pltpu.sync_copy(f_sk.at[pl.ds(tok0, C_TOK)], buf32)
            pltpu.sync_copy(buf32, dst.at[:, pl.ds(SK_COL, 32)])
            return carry

        lax.fori_loop(0, n_chunks, chunk, 0)

    return sc_pack


def _tc_body(e, f, bfold, gam, bet, out):
    acc = jnp.dot(e[...], f[...], preferred_element_type=jnp.float32)
    acc += bfold[...]
    val = acc[:, :D_OUT]
    gate = jax.nn.sigmoid(acc[:, D_OUT:])
    z = gate * val
    mu = jnp.mean(z, axis=1, keepdims=True)
    zc = z - mu
    var = jnp.mean(zc * zc, axis=1, keepdims=True)
    out[...] = zc / jnp.sqrt(var + 1e-5) * gam[...] + bet[...]


def _tc_call(n_tokens, tn, args):
    grid = (n_tokens // tn,)
    in_specs = [
        pl.BlockSpec((tn, 128), lambda i: (i, 0)),
        pl.BlockSpec((128, 256), lambda i: (0, 0)),
        pl.BlockSpec((1, 256), lambda i: (0, 0)),
        pl.BlockSpec((1, D_OUT), lambda i: (0, 0)),
        pl.BlockSpec((1, D_OUT), lambda i: (0, 0)),
    ]
    return pl.pallas_call(
        _tc_body,
        grid=grid,
        in_specs=in_specs,
        out_specs=pl.BlockSpec((tn, D_OUT), lambda i: (i, 0)),
        out_shape=jax.ShapeDtypeStruct((n_tokens, D_OUT), jnp.float32),
        compiler_params=pltpu.CompilerParams(
            dimension_semantics=("arbitrary",)),
    )(*args)


def kernel(low8, high8, alignment, small_int, delta, hamming,
           continuous, bitfield, sketch,
           low8_table, high8_table, align_table, small_table, delta_table,
           hamming_table,
           W_cont, b_cont, W_bits, b_bits, W_sketch, b_sketch,
           W_val, b_val, W_gate, b_gate, ln_gamma, ln_beta):
    B, L = low8.shape
    n = B * L

    def padw(t, w):
        return jnp.pad(t, ((0, 0), (0, w - t.shape[1])))

    raw_tables = (low8_table, high8_table, align_table, small_table,
                  delta_table, hamming_table)
    tables = tuple(padw(t, w) for t, (_, w) in zip(raw_tables, EMB_FIELDS))
    idxs = tuple(
        a.reshape(n)
        for a in (low8, high8, alignment, small_int, delta, hamming))
    f_cont = jnp.pad(continuous.reshape(n, 6), ((0, 0), (0, 2)))
    f_bits = bitfield.reshape(n, 16)
    f_sk = sketch.reshape(n, 32)

    e_all = _make_sc_pack(n)(*tables, *idxs, f_cont, f_bits, f_sk)

    qc = _ternary_fwd(W_cont)
    qb = _ternary_fwd(W_bits)
    qs = _ternary_fwd(W_sketch)
    Wvg = jnp.concatenate([W_val, W_gate], axis=0)

    f = jnp.zeros((128, 256), jnp.float32)
    spans = ((0, 8), (8, 16), (16, 20), (20, 24), (24, 30), (30, 34))
    for (c0, c1), (col, _) in zip(spans, EMB_FIELDS):
        f = f.at[col:col + (c1 - c0), :].set(Wvg[:, c0:c1].T)
    f = f.at[CONT_COL:CONT_COL + 6, :].set(qc.T @ Wvg[:, 34:42].T)
    f = f.at[BITS_COL:BITS_COL + 16, :].set(qb.T @ Wvg[:, 42:50].T)
    f = f.at[SK_COL:SK_COL + 32, :].set(qs.T @ Wvg[:, 50:62].T)
    bfold = (jnp.concatenate([b_val, b_gate])
             + Wvg[:, 34:42] @ b_cont
             + Wvg[:, 42:50] @ b_bits
             + Wvg[:, 50:62] @ b_sketch)[None, :]

    out = _tc_call(
        n, 1024,
        (e_all, f, bfold, ln_gamma[None, :], ln_beta[None, :]))
    return out.reshape(B, L, D_OUT)


# speedup vs baseline: 5.3337x; 1.3471x over previous
import functools

import jax
import jax.numpy as jnp
from jax import lax
from jax.experimental import pallas as pl
from jax.experimental.pallas import tpu as pltpu
from jax.experimental.pallas import tpu_sc as plsc

D_OUT = 128
NW = 32
CB = 16
L_SEQ = 50

EMB_FIELDS = ((0, 8), (8, 8), (16, 8), (24, 8), (32, 8), (40, 8))
FEAT_FIELDS = ((48, 6), (56, 16), (72, 32))


def _ternary_fwd(w):
    scale = jnp.mean(jnp.abs(w)) + 1e-5
    return jnp.clip(jnp.round(w / scale), -1.0, 1.0) * scale


@functools.lru_cache(maxsize=None)
def _make_sc_pack(n_b):
    rows_w = n_b // NW
    n_chunks = rows_w // CB
    c_tok = CB * L_SEQ

    mesh = plsc.VectorSubcoreMesh(core_axis_name="c", subcore_axis_name="s")

    @functools.partial(
        pl.kernel,
        mesh=mesh,
        out_type=jax.ShapeDtypeStruct((n_b * L_SEQ, 128), jnp.float32),
        compiler_params=pltpu.CompilerParams(use_tc_tiling_on_sc=False),
        scratch_types=[
            [pltpu.VMEM((CB, L_SEQ), jnp.int32) for _ in range(6)],
            [pltpu.VMEM((c_tok, 8), jnp.float32) for _ in range(6)],
            [pltpu.VMEM((c_tok, w), jnp.float32) for _, w in FEAT_FIELDS],
            pltpu.SemaphoreType.DMA,
        ],
    )
    def sc_pack(t_low, t_high, t_align, t_small, t_delta, t_ham,
                i_low, i_high, i_align, i_small, i_delta, i_ham,
                f_cont, f_bits, f_sk,
                e_all, idx_bufs, gbufs, fbufs, sem):
        wid = lax.axis_index("c") * 16 + lax.axis_index("s")
        b_base = wid * rows_w

        tables = (t_low, t_high, t_align, t_small, t_delta, t_ham)
        idxs = (i_low, i_high, i_align, i_small, i_delta, i_ham)
        feats = (f_cont, f_bits, f_sk)

        def chunk(j, carry):
            b0 = b_base + j * CB
            tok0 = b0 * L_SEQ
            cps = [pltpu.async_copy(idx.at[pl.ds(b0, CB)], buf, sem)
                   for idx, buf in zip(idxs, idx_bufs)]
            for cp in cps:
                cp.wait()
            cps = []
            for table, ibuf, gbuf in zip(tables, idx_bufs, gbufs):
                for r in range(CB):
                    cps.append(pltpu.async_copy(
                        table.at[ibuf.at[r]],
                        gbuf.at[pl.ds(r * L_SEQ, L_SEQ)], sem))
            for feat, fbuf in zip(feats, fbufs):
                for r in range(CB):
                    cps.append(pltpu.async_copy(
                        feat.at[b0 + r],
                        fbuf.at[pl.ds(r * L_SEQ, L_SEQ)], sem))
            for cp in cps:
                cp.wait()
            dst = e_all.at[pl.ds(tok0, c_tok)]
            cps = [pltpu.async_copy(gbuf, dst.at[:, pl.ds(col, w)], sem)
                   for gbuf, (col, w) in zip(gbufs, EMB_FIELDS)]
            cps += [pltpu.async_copy(fbuf, dst.at[:, pl.ds(col, w)], sem)
                    for fbuf, (col, w) in zip(fbufs, FEAT_FIELDS)]
            for cp in cps:
                cp.wait()
            return carry

        lax.fori_loop(0, n_chunks, chunk, 0)

    return sc_pack


def _tc_body(e, f, mask, bfold, gam, bet, out):
    x = jnp.where(mask[...] != 0, e[...], 0.0)
    acc = jnp.dot(x, f[...], preferred_element_type=jnp.float32)
    acc += bfold[...]
    val = acc[:, :D_OUT]
    gate = jax.nn.sigmoid(acc[:, D_OUT:])
    z = gate * val
    mu = jnp.mean(z, axis=1, keepdims=True)
    zc = z - mu
    var = jnp.mean(zc * zc, axis=1, keepdims=True)
    res = zc / jnp.sqrt(var + 1e-5) * gam[...] + bet[...]
    for g in range(CB):
        out[g, :, :] = res[g * L_SEQ:(g + 1) * L_SEQ, :]


def _tc_call(n_b, args):
    c_tok = CB * L_SEQ
    grid = (n_b // CB,)
    in_specs = [
        pl.BlockSpec((c_tok, 128), lambda i: (i, 0)),
        pl.BlockSpec((128, 256), lambda i: (0, 0)),
        pl.BlockSpec((1, 128), lambda i: (0, 0)),
        pl.BlockSpec((1, 256), lambda i: (0, 0)),
        pl.BlockSpec((1, D_OUT), lambda i: (0, 0)),
        pl.BlockSpec((1, D_OUT), lambda i: (0, 0)),
    ]
    return pl.pallas_call(
        _tc_body,
        grid=grid,
        in_specs=in_specs,
        out_specs=pl.BlockSpec((CB, L_SEQ, D_OUT), lambda i: (i, 0, 0)),
        out_shape=jax.ShapeDtypeStruct((n_b, L_SEQ, D_OUT), jnp.float32),
        compiler_params=pltpu.CompilerParams(
            dimension_semantics=("arbitrary",)),
    )(*args)


def kernel(low8, high8, alignment, small_int, delta, hamming,
           continuous, bitfield, sketch,
           low8_table, high8_table, align_table, small_table, delta_table,
           hamming_table,
           W_cont, b_cont, W_bits, b_bits, W_sketch, b_sketch,
           W_val, b_val, W_gate, b_gate, ln_gamma, ln_beta):
    B, L = low8.shape
    n = B * L

    def pad8(t):
        return jnp.pad(t, ((0, 0), (0, 8 - t.shape[1])))

    tables = (low8_table, high8_table, pad8(align_table), pad8(small_table),
              pad8(delta_table), pad8(hamming_table))

    e_all = _make_sc_pack(B)(*tables, low8, high8, alignment, small_int,
                             delta, hamming, continuous, bitfield, sketch)

    qc = _ternary_fwd(W_cont)
    qb = _ternary_fwd(W_bits)
    qs = _ternary_fwd(W_sketch)
    Wvg = jnp.concatenate([W_val, W_gate], axis=0)

    f = jnp.zeros((128, 256), jnp.float32)
    spans = ((0, 8), (8, 16), (16, 20), (20, 24), (24, 30), (30, 34))
    for (c0, c1), (col, _) in zip(spans, EMB_FIELDS):
        f = f.at[col:col + (c1 - c0), :].set(Wvg[:, c0:c1].T)
    f = f.at[48:54, :].set(qc.T @ Wvg[:, 34:42].T)
    f = f.at[56:72, :].set(qb.T @ Wvg[:, 42:50].T)
    f = f.at[72:104, :].set(qs.T @ Wvg[:, 50:62].T)
    bfold = (jnp.concatenate([b_val, b_gate])
             + Wvg[:, 34:42] @ b_cont
             + Wvg[:, 42:50] @ b_bits
             + Wvg[:, 50:62] @ b_sketch)[None, :]

    mask = jnp.zeros((1, 128), jnp.float32)
    mask = mask.at[:, 0:54].set(1.0).at[:, 56:104].set(1.0)

    return _tc_call(
        B, (e_all, f, mask, bfold, ln_gamma[None, :], ln_beta[None, :]))


# speedup vs baseline: 7.1906x; 1.3481x over previous
import functools

import jax
import jax.numpy as jnp
from jax import lax
from jax.experimental import pallas as pl
from jax.experimental.pallas import tpu as pltpu
from jax.experimental.pallas import tpu_sc as plsc

D_OUT = 128
NW = 32
CB = 16
L_SEQ = 50

EMB_FIELDS = ((0, 8), (8, 8), (16, 8), (24, 8), (32, 8), (40, 8))


def _ternary_fwd(w):
    scale = jnp.mean(jnp.abs(w)) + 1e-5
    return jnp.clip(jnp.round(w / scale), -1.0, 1.0) * scale


@functools.lru_cache(maxsize=None)
def _make_sc_pack(n_b):
    rows_w = n_b // NW
    n_chunks = rows_w // CB
    c_tok = CB * L_SEQ

    mesh = plsc.VectorSubcoreMesh(core_axis_name="c", subcore_axis_name="s")

    @functools.partial(
        pl.kernel,
        mesh=mesh,
        out_type=jax.ShapeDtypeStruct((n_b * L_SEQ, 128), jnp.float32),
        compiler_params=pltpu.CompilerParams(use_tc_tiling_on_sc=False),
        scratch_types=[
            [pltpu.VMEM((CB, L_SEQ), jnp.int32) for _ in range(6)],
            [pltpu.VMEM((c_tok, 8), jnp.float32) for _ in range(6)],
            pltpu.SemaphoreType.DMA,
        ],
    )
    def sc_pack(t_low, t_high, t_align, t_small, t_delta, t_ham,
                i_low, i_high, i_align, i_small, i_delta, i_ham,
                e_all, idx_bufs, gbufs, sem):
        wid = lax.axis_index("c") * 16 + lax.axis_index("s")
        b_base = wid * rows_w

        tables = (t_low, t_high, t_align, t_small, t_delta, t_ham)
        idxs = (i_low, i_high, i_align, i_small, i_delta, i_ham)

        def chunk(j, carry):
            b0 = b_base + j * CB
            tok0 = b0 * L_SEQ
            cps = [pltpu.async_copy(idx.at[pl.ds(b0, CB)], buf, sem)
                   for idx, buf in zip(idxs, idx_bufs)]
            for cp in cps:
                cp.wait()
            cps = []
            for table, ibuf, gbuf in zip(tables, idx_bufs, gbufs):
                for r in range(CB):
                    cps.append(pltpu.async_copy(
                        table.at[ibuf.at[r]],
                        gbuf.at[pl.ds(r * L_SEQ, L_SEQ)], sem))
            for cp in cps:
                cp.wait()
            dst = e_all.at[pl.ds(tok0, c_tok)]
            cps = [pltpu.async_copy(gbuf, dst.at[:, pl.ds(col, w)], sem)
                   for gbuf, (col, w) in zip(gbufs, EMB_FIELDS)]
            for cp in cps:
                cp.wait()
            return carry

        lax.fori_loop(0, n_chunks, chunk, 0)

    return sc_pack


def _tc_body(e, cont, bits, sk, f, fc, fb, fs, mask, bfold, gam, bet, out):
    x = jnp.where(mask[...] != 0, e[...], 0.0)
    acc = jnp.dot(x, f[...], preferred_element_type=jnp.float32)
    xc = jnp.concatenate([cont[g] for g in range(CB)], axis=0)
    acc += jnp.dot(xc, fc[...], preferred_element_type=jnp.float32)
    xb = jnp.concatenate([bits[g] for g in range(CB)], axis=0)
    acc += jnp.dot(xb, fb[...], preferred_element_type=jnp.float32)
    xs = jnp.concatenate([sk[g] for g in range(CB)], axis=0)
    acc += jnp.dot(xs, fs[...], preferred_element_type=jnp.float32)
    acc += bfold[...]
    val = acc[:, :D_OUT]
    gate = jax.nn.sigmoid(acc[:, D_OUT:])
    z = gate * val
    mu = jnp.mean(z, axis=1, keepdims=True)
    zc = z - mu
    var = jnp.mean(zc * zc, axis=1, keepdims=True)
    res = zc / jnp.sqrt(var + 1e-5) * gam[...] + bet[...]
    for g in range(CB):
        out[g, :, :] = res[g * L_SEQ:(g + 1) * L_SEQ, :]


def _tc_call(n_b, args):
    c_tok = CB * L_SEQ
    grid = (n_b // CB,)
    in_specs = [
        pl.BlockSpec((c_tok, 128), lambda i: (i, 0)),
        pl.BlockSpec((CB, L_SEQ, 6), lambda i: (i, 0, 0)),
        pl.BlockSpec((CB, L_SEQ, 16), lambda i: (i, 0, 0)),
        pl.BlockSpec((CB, L_SEQ, 32), lambda i: (i, 0, 0)),
        pl.BlockSpec((128, 256), lambda i: (0, 0)),
        pl.BlockSpec((6, 256), lambda i: (0, 0)),
        pl.BlockSpec((16, 256), lambda i: (0, 0)),
        pl.BlockSpec((32, 256), lambda i: (0, 0)),
        pl.BlockSpec((1, 128), lambda i: (0, 0)),
        pl.BlockSpec((1, 256), lambda i: (0, 0)),
        pl.BlockSpec((1, D_OUT), lambda i: (0, 0)),
        pl.BlockSpec((1, D_OUT), lambda i: (0, 0)),
    ]
    return pl.pallas_call(
        _tc_body,
        grid=grid,
        in_specs=in_specs,
        out_specs=pl.BlockSpec((CB, L_SEQ, D_OUT), lambda i: (i, 0, 0)),
        out_shape=jax.ShapeDtypeStruct((n_b, L_SEQ, D_OUT), jnp.float32),
        compiler_params=pltpu.CompilerParams(
            dimension_semantics=("arbitrary",)),
    )(*args)


def kernel(low8, high8, alignment, small_int, delta, hamming,
           continuous, bitfield, sketch,
           low8_table, high8_table, align_table, small_table, delta_table,
           hamming_table,
           W_cont, b_cont, W_bits, b_bits, W_sketch, b_sketch,
           W_val, b_val, W_gate, b_gate, ln_gamma, ln_beta):
    B, L = low8.shape
    n = B * L

    def pad8(t):
        return jnp.pad(t, ((0, 0), (0, 8 - t.shape[1])))

    tables = (low8_table, high8_table, pad8(align_table), pad8(small_table),
              pad8(delta_table), pad8(hamming_table))

    e_all = _make_sc_pack(B)(*tables, low8, high8, alignment, small_int,
                             delta, hamming)

    qc = _ternary_fwd(W_cont)
    qb = _ternary_fwd(W_bits)
    qs = _ternary_fwd(W_sketch)
    Wvg = jnp.concatenate([W_val, W_gate], axis=0)

    f = jnp.zeros((128, 256), jnp.float32)
    spans = ((0, 8), (8, 16), (16, 20), (20, 24), (24, 30), (30, 34))
    for (c0, c1), (col, _) in zip(spans, EMB_FIELDS):
        f = f.at[col:col + (c1 - c0), :].set(Wvg[:, c0:c1].T)
    fc = qc.T @ Wvg[:, 34:42].T
    fb = qb.T @ Wvg[:, 42:50].T
    fs = qs.T @ Wvg[:, 50:62].T
    bfold = (jnp.concatenate([b_val, b_gate])
             + Wvg[:, 34:42] @ b_cont
             + Wvg[:, 42:50] @ b_bits
             + Wvg[:, 50:62] @ b_sketch)[None, :]

    mask = jnp.zeros((1, 128), jnp.float32).at[:, 0:48].set(1.0)

    return _tc_call(
        B, (e_all, continuous, bitfield, sketch, f, fc, fb, fs, mask,
            bfold, ln_gamma[None, :], ln_beta[None, :]))
